# baseline (speedup 1.0000x reference)
import functools

import jax
import jax.numpy as jnp
from jax import lax
from jax.experimental import pallas as pl
from jax.experimental.pallas import tpu as pltpu
from jax.experimental.pallas import tpu_sc as plsc

B, F, D = 4096, 26, 16
NW = 32
BPW = B // NW
NL = 16
NGRP = BPW // NL

_mesh = plsc.VectorSubcoreMesh(core_axis_name="c", subcore_axis_name="s")


@functools.partial(
    pl.kernel,
    mesh=_mesh,
    out_type=jax.ShapeDtypeStruct((B,), jnp.float32),
    compiler_params=pltpu.CompilerParams(needs_layout_passes=False),
    scratch_types=[
        pltpu.VMEM((2, F // 2, D, BPW), jnp.float32),
        pltpu.VMEM((NGRP, D, NL), jnp.float32),
        pltpu.VMEM((NGRP, D, NL), jnp.float32),
        pltpu.VMEM((BPW,), jnp.float32),
        pltpu.SemaphoreType.DMA,
        pltpu.SemaphoreType.DMA,
    ],
)
def _fm_dense(emb_hbm, sec_hbm, emb_v, s_v, q_v, sec_v, sem_e0, sem_e1):
    wid = lax.axis_index("s") * 2 + lax.axis_index("c")
    b0 = wid * BPW
    HF = F // 2
    cps = [
        pltpu.async_copy(emb_hbm.at[pl.ds(h * HF, HF), :, pl.ds(b0, BPW)],
                         emb_v.at[h], sem)
        for h, sem in ((0, sem_e0), (1, sem_e1))
    ]

    cps[0].wait()
    for g in range(NGRP):
        gb = g * NL

        def body0(d, _, g=g, gb=gb):
            v0 = emb_v[0, 0, d, pl.ds(gb, NL)]
            acc_s = v0
            acc_q = v0 * v0
            for f in range(1, HF):
                v = emb_v[0, f, d, pl.ds(gb, NL)]
                acc_s = acc_s + v
                acc_q = acc_q + v * v
            s_v[g, d, :] = acc_s
            q_v[g, d, :] = acc_q
            return 0

        lax.fori_loop(0, D, body0, 0)

    cps[1].wait()
    for g in range(NGRP):
        gb = g * NL

        def body1(d, sec, g=g, gb=gb):
            v0 = emb_v[1, 0, d, pl.ds(gb, NL)]
            acc_s = v0
            acc_q = v0 * v0
            for f in range(1, HF):
                v = emb_v[1, f, d, pl.ds(gb, NL)]
                acc_s = acc_s + v
                acc_q = acc_q + v * v
            acc_s = acc_s + s_v[g, d, :]
            acc_q = acc_q + q_v[g, d, :]
            return sec + (acc_s * acc_s - acc_q)

        sec = lax.fori_loop(0, D, body1, jnp.zeros((NL,), jnp.float32))
        sec_v[pl.ds(gb, NL)] = 0.5 * sec

    pltpu.sync_copy(sec_v, sec_hbm.at[pl.ds(b0, BPW)])


@functools.partial(
    pl.kernel,
    mesh=_mesh,
    out_type=jax.ShapeDtypeStruct((B,), jnp.float32),
    compiler_params=pltpu.CompilerParams(needs_layout_passes=False),
    scratch_types=[
        pltpu.VMEM((F, BPW), jnp.int32),
        pltpu.VMEM((F * BPW,), jnp.float32),
        pltpu.VMEM((BPW,), jnp.float32),
        pltpu.SemaphoreType.DMA,
    ],
)
def _fm_gather(idx_hbm, w_hbm, out_hbm, idx_v, rows_v, out_v, sem_g):
    wid = lax.axis_index("s") * 2 + lax.axis_index("c")
    b0 = wid * BPW

    pltpu.sync_copy(idx_hbm.at[:, pl.ds(b0, BPW)], idx_v)
    cps_g = [
        pltpu.async_copy(w_hbm.at[idx_v.at[f]],
                         rows_v.at[pl.ds(f * BPW, BPW)], sem_g)
        for f in range(F)
    ]
    for cp in cps_g:
        cp.wait()

    for g in range(NGRP):
        gb = g * NL
        first = rows_v[pl.ds(gb, NL)]
        for f in range(1, F):
            first = first + rows_v[pl.ds(f * BPW + gb, NL)]
        out_v[pl.ds(gb, NL)] = first

    pltpu.sync_copy(out_v, out_hbm.at[pl.ds(b0, BPW)])


def kernel(sparse_inputs, embed_inputs, w):
    idx = sparse_inputs.astype(jnp.int32).T
    emb = jnp.transpose(embed_inputs, (1, 2, 0))
    wf = jnp.pad(w.T, ((0, 0), (0, 448))).reshape(-1)
    sec = _fm_dense(emb)
    first = _fm_gather(idx, wf)
    return (first + sec).reshape(B, 1)


# speedup vs baseline: 1.1844x; 1.1844x over previous
import functools

import jax
import jax.numpy as jnp
from jax import lax
from jax.experimental import pallas as pl
from jax.experimental.pallas import tpu as pltpu
from jax.experimental.pallas import tpu_sc as plsc

B, F, D = 4096, 26, 16
NW = 32
BPW = B // NW
NL = 16
NGRP = BPW // NL

_mesh = plsc.VectorSubcoreMesh(core_axis_name="c", subcore_axis_name="s")


@functools.partial(
    pl.kernel,
    mesh=_mesh,
    out_type=jax.ShapeDtypeStruct((B,), jnp.float32),
    compiler_params=pltpu.CompilerParams(needs_layout_passes=False),
    scratch_types=[
        pltpu.VMEM((2, F // 2, D, BPW), jnp.float32),
        pltpu.VMEM((NGRP, D, NL), jnp.float32),
        pltpu.VMEM((NGRP, D, NL), jnp.float32),
        pltpu.VMEM((BPW,), jnp.float32),
        pltpu.SemaphoreType.DMA,
        pltpu.SemaphoreType.DMA,
    ],
)
def _fm_dense(emb_hbm, sec_hbm, emb_v, s_v, q_v, sec_v, sem_e0, sem_e1):
    wid = lax.axis_index("s") * 2 + lax.axis_index("c")
    b0 = wid * BPW
    HF = F // 2
    cps = [
        pltpu.async_copy(emb_hbm.at[pl.ds(h * HF, HF), :, pl.ds(b0, BPW)],
                         emb_v.at[h], sem)
        for h, sem in ((0, sem_e0), (1, sem_e1))
    ]

    cps[0].wait()
    for g in range(NGRP):
        gb = g * NL

        def body0(d, _, g=g, gb=gb):
            v0 = emb_v[0, 0, d, pl.ds(gb, NL)]
            acc_s = v0
            acc_q = v0 * v0
            for f in range(1, HF):
                v = emb_v[0, f, d, pl.ds(gb, NL)]
                acc_s = acc_s + v
                acc_q = acc_q + v * v
            s_v[g, d, :] = acc_s
            q_v[g, d, :] = acc_q
            return 0

        lax.fori_loop(0, D, body0, 0)

    cps[1].wait()
    for g in range(NGRP):
        gb = g * NL

        def body1(d, sec, g=g, gb=gb):
            v0 = emb_v[1, 0, d, pl.ds(gb, NL)]
            acc_s = v0
            acc_q = v0 * v0
            for f in range(1, HF):
                v = emb_v[1, f, d, pl.ds(gb, NL)]
                acc_s = acc_s + v
                acc_q = acc_q + v * v
            acc_s = acc_s + s_v[g, d, :]
            acc_q = acc_q + q_v[g, d, :]
            return sec + (acc_s * acc_s - acc_q)

        sec = lax.fori_loop(0, D, body1, jnp.zeros((NL,), jnp.float32))
        sec_v[pl.ds(gb, NL)] = 0.5 * sec

    pltpu.sync_copy(sec_v, sec_hbm.at[pl.ds(b0, BPW)])


@functools.partial(
    pl.kernel,
    mesh=_mesh,
    out_type=jax.ShapeDtypeStruct((B,), jnp.float32),
    compiler_params=pltpu.CompilerParams(needs_layout_passes=False),
    scratch_types=[
        pltpu.VMEM((F, BPW), jnp.int32),
        pltpu.VMEM((F * BPW,), jnp.float32),
        pltpu.VMEM((BPW,), jnp.float32),
        pltpu.VMEM((BPW,), jnp.float32),
        pltpu.SemaphoreType.DMA,
        pltpu.SemaphoreType.DMA,
    ],
)
def _fm_gather(idx_hbm, w_hbm, sec_hbm, out_hbm, idx_v, rows_v, sec_v, out_v,
               sem_g, sem_s):
    wid = lax.axis_index("s") * 2 + lax.axis_index("c")
    b0 = wid * BPW

    cp_s = pltpu.async_copy(sec_hbm.at[pl.ds(b0, BPW)], sec_v, sem_s)
    pltpu.sync_copy(idx_hbm.at[:, pl.ds(b0, BPW)], idx_v)
    cps_g = [
        pltpu.async_copy(w_hbm.at[idx_v.at[f]],
                         rows_v.at[pl.ds(f * BPW, BPW)], sem_g)
        for f in range(F)
    ]
    for cp in cps_g:
        cp.wait()
    cp_s.wait()

    for g in range(NGRP):
        gb = g * NL
        first = rows_v[pl.ds(gb, NL)]
        for f in range(1, F):
            first = first + rows_v[pl.ds(f * BPW + gb, NL)]
        out_v[pl.ds(gb, NL)] = sec_v[pl.ds(gb, NL)] + first

    pltpu.sync_copy(out_v, out_hbm.at[pl.ds(b0, BPW)])


def kernel(sparse_inputs, embed_inputs, w):
    idx = sparse_inputs.astype(jnp.int32).T
    emb = jnp.transpose(embed_inputs, (1, 2, 0))
    wf = jnp.pad(w.T, ((0, 0), (0, 448))).reshape(-1)
    sec = _fm_dense(emb)
    out = _fm_gather(idx, wf, sec)
    return out.reshape(B, 1)


# speedup vs baseline: 1.2795x; 1.0803x over previous
import functools

import jax
import jax.numpy as jnp
from jax import lax
from jax.experimental import pallas as pl
from jax.experimental.pallas import tpu as pltpu
from jax.experimental.pallas import tpu_sc as plsc

B, F, D = 4096, 26, 16
NW = 32
BPW = B // NW
NL = 16
NGRP = BPW // NL

_mesh = plsc.VectorSubcoreMesh(core_axis_name="c", subcore_axis_name="s")


@functools.partial(
    pl.kernel,
    mesh=_mesh,
    out_type=jax.ShapeDtypeStruct((B,), jnp.float32),
    compiler_params=pltpu.CompilerParams(needs_layout_passes=False),
    scratch_types=[
        pltpu.VMEM((F, BPW), jnp.int32),
        pltpu.VMEM((F * BPW,), jnp.float32),
        pltpu.VMEM((2, F // 2, D, BPW), jnp.float32),
        pltpu.VMEM((NGRP, D, NL), jnp.float32),
        pltpu.VMEM((NGRP, D, NL), jnp.float32),
        pltpu.VMEM((BPW,), jnp.float32),
        pltpu.SemaphoreType.DMA,
        pltpu.SemaphoreType.DMA,
        pltpu.SemaphoreType.DMA,
    ],
)
def _fm_sc(idx_hbm, emb_hbm, w_hbm, out_hbm, idx_v, rows_v, emb_v, s_v, q_v,
           out_v, sem_g, sem_e0, sem_e1):
    wid = lax.axis_index("s") * 2 + lax.axis_index("c")
    b0 = wid * BPW
    HF = F // 2

    cps_e = [
        pltpu.async_copy(emb_hbm.at[pl.ds(h * HF, HF), :, pl.ds(b0, BPW)],
                         emb_v.at[h], sem)
        for h, sem in ((0, sem_e0), (1, sem_e1))
    ]
    pltpu.sync_copy(idx_hbm.at[:, pl.ds(b0, BPW)], idx_v)
    cps_g = [
        pltpu.async_copy(w_hbm.at[idx_v.at[f]],
                         rows_v.at[pl.ds(f * BPW, BPW)], sem_g)
        for f in range(F)
    ]

    cps_e[0].wait()
    for g in range(NGRP):
        gb = g * NL

        def body0(d, _, g=g, gb=gb):
            v0 = emb_v[0, 0, d, pl.ds(gb, NL)]
            acc_s = v0
            acc_q = v0 * v0
            for f in range(1, HF):
                v = emb_v[0, f, d, pl.ds(gb, NL)]
                acc_s = acc_s + v
                acc_q = acc_q + v * v
            s_v[g, d, :] = acc_s
            q_v[g, d, :] = acc_q
            return 0

        lax.fori_loop(0, D, body0, 0)

    cps_e[1].wait()
    for g in range(NGRP):
        gb = g * NL

        def body1(d, sec, g=g, gb=gb):
            v0 = emb_v[1, 0, d, pl.ds(gb, NL)]
            acc_s = v0
            acc_q = v0 * v0
            for f in range(1, HF):
                v = emb_v[1, f, d, pl.ds(gb, NL)]
                acc_s = acc_s + v
                acc_q = acc_q + v * v
            acc_s = acc_s + s_v[g, d, :]
            acc_q = acc_q + q_v[g, d, :]
            return sec + (acc_s * acc_s - acc_q)

        sec = lax.fori_loop(0, D, body1, jnp.zeros((NL,), jnp.float32))
        out_v[pl.ds(gb, NL)] = 0.5 * sec

    for cp in cps_g:
        cp.wait()
    for g in range(NGRP):
        gb = g * NL
        first = rows_v[pl.ds(gb, NL)]
        for f in range(1, F):
            first = first + rows_v[pl.ds(f * BPW + gb, NL)]
        out_v[pl.ds(gb, NL)] = out_v[pl.ds(gb, NL)] + first

    pltpu.sync_copy(out_v, out_hbm.at[pl.ds(b0, BPW)])


def kernel(sparse_inputs, embed_inputs, w):
    idx = sparse_inputs.astype(jnp.int32).T
    emb = jnp.transpose(embed_inputs, (1, 2, 0))
    wf = jnp.pad(w.T, ((0, 0), (0, 448))).reshape(-1)
    out = _fm_sc(idx, emb, wf)
    return out.reshape(B, 1)
